# Initial kernel scaffold; baseline (speedup 1.0000x reference)
import functools
import jax
import jax.numpy as jnp
from jax import lax
from jax.experimental import pallas as pl
from jax.experimental.pallas import tpu as pltpu, tpu_sc as plsc

B, C, HP, WP = 2, 96, 192, 192
H, W = 384, 384
NPLANE = B * C
NUP = HP * WP
MOUT = H * W
HALF = MOUT // 2
CH = 4608
NCHUNK = NUP // CH
VPC = CH // 16

NC, NS = 2, 16
NW = NC * NS
PPW = NPLANE // NW

_mesh = plsc.VectorSubcoreMesh(core_axis_name="c", subcore_axis_name="s")


@functools.partial(
    pl.kernel,
    out_type=jax.ShapeDtypeStruct((NPLANE * MOUT,), jnp.float32),
    mesh=_mesh,
    scratch_types=[
        pltpu.VMEM((HALF,), jnp.float32),
        pltpu.VMEM((CH,), jnp.int32),
        pltpu.VMEM((CH,), jnp.float32),
    ],
)
def _unpool(f_hbm, prov_hbm, out_hbm, buf, idx_v, val_v):
  wid = lax.axis_index("s") * NC + lax.axis_index("c")
  zeros16 = jnp.zeros((16,), jnp.float32)

  @pl.loop(0, PPW)
  def _plane(p):
    plane = wid * PPW + p
    up_base = plane * NUP
    out_base = plane * MOUT
    for half in range(2):
      lo = half * HALF

      @pl.loop(0, HALF // 16)
      def _zero(i):
        buf[pl.ds(i * 16, 16)] = zeros16

      @pl.loop(0, NCHUNK)
      def _chunk(c):
        off = up_base + c * CH
        pltpu.sync_copy(prov_hbm.at[pl.ds(off, CH)], idx_v)
        pltpu.sync_copy(f_hbm.at[pl.ds(off, CH)], val_v)

        @pl.loop(0, VPC)
        def _vec(i):
          idxv = idx_v[pl.ds(i * 16, 16)]
          valv = val_v[pl.ds(i * 16, 16)]
          local = idxv - lo
          mask = (local >= 0) & (local < HALF)
          safe = jnp.where(mask, local, 0)
          plsc.store_scatter(buf, [safe], valv, mask=mask)

      pltpu.sync_copy(buf, out_hbm.at[pl.ds(out_base + lo, HALF)])


def kernel(f, provenance):
  out = _unpool(f.reshape(-1), provenance.reshape(-1))
  return out.reshape(B, C, H, W)


# speedup vs baseline: 4.3274x; 4.3274x over previous
import functools
import jax
import jax.numpy as jnp
from jax import lax
from jax.experimental import pallas as pl
from jax.experimental.pallas import tpu as pltpu, tpu_sc as plsc

B, C, HP, WP = 2, 96, 192, 192
H, W = 384, 384
NPLANE = B * C
NUP = HP * WP
MOUT = H * W
HALF = MOUT // 2
NTOT = NPLANE * NUP
WLEN = 19968
WOFF = NUP - WLEN
CH = 4992
NCHUNK = WLEN // CH
VPC = CH // 16

NC, NS = 2, 16
NW = NC * NS
PPW = NPLANE // NW

_mesh = plsc.VectorSubcoreMesh(core_axis_name="c", subcore_axis_name="s")


@functools.partial(
    pl.kernel,
    out_type=jax.ShapeDtypeStruct((NPLANE * MOUT,), jnp.float32),
    mesh=_mesh,
    scratch_types=[
        pltpu.VMEM((HALF,), jnp.float32),
        pltpu.VMEM((CH + 16,), jnp.int32),
        pltpu.VMEM((CH,), jnp.float32),
    ],
    compiler_params=pltpu.CompilerParams(needs_layout_passes=False),
)
def _scatter_sorted(sk_hbm, sv_hbm, out_hbm, buf, kv, vv):
  wid = lax.axis_index("s") * NC + lax.axis_index("c")
  zeros16 = jnp.zeros((16,), jnp.float32)
  lane = jax.lax.iota(jnp.int32, 16)

  @pl.loop(0, PPW)
  def _plane(p):
    plane = wid * PPW + p
    p0 = plane * NUP
    for half in range(2):
      wstart = p0 + half * WOFF
      kbase = plane * MOUT + half * HALF

      @pl.loop(0, HALF // 16)
      def _zero(i):
        buf[pl.ds(i * 16, 16)] = zeros16

      @pl.loop(0, NCHUNK)
      def _chunk(c):
        start = wstart + c * CH
        pltpu.sync_copy(sk_hbm.at[pl.ds(start, CH)], kv.at[pl.ds(0, CH)])
        la = jnp.minimum(start + CH, NTOT - 16)
        pltpu.sync_copy(sk_hbm.at[pl.ds(la, 16)], kv.at[pl.ds(CH, 16)])
        pltpu.sync_copy(sv_hbm.at[pl.ds(start, CH)], vv)

        @pl.loop(0, VPC)
        def _vec(i):
          k0 = kv[pl.ds(i * 16, 16)]
          k1 = kv[pl.ds(i * 16 + 1, 16)]
          v = vv[pl.ds(i * 16, 16)]
          local = k0 - kbase
          inr = plsc.bitcast(local, jnp.uint32) < jnp.uint32(HALF)
          pos = (start + i * 16) + lane
          keep = (k0 != k1) | (pos == NTOT - 1)
          m = inr & keep
          plsc.store_scatter(buf, [local], v, mask=m)

      pltpu.sync_copy(buf, out_hbm.at[pl.ds(kbase, HALF)])


def kernel(f, provenance):
  plane_off = (jnp.arange(NPLANE, dtype=jnp.int32) * MOUT).reshape(B, C, 1)
  keys = (provenance.reshape(B, C, NUP) + plane_off).reshape(-1)
  sk, sv = lax.sort((keys, f.reshape(-1)), dimension=0, is_stable=False,
                    num_keys=1)
  out = _scatter_sorted(sk, sv)
  return out.reshape(B, C, H, W)


# speedup vs baseline: 4.3814x; 1.0125x over previous
import functools
import jax
import jax.numpy as jnp
from jax import lax
from jax.experimental import pallas as pl
from jax.experimental.pallas import tpu as pltpu, tpu_sc as plsc

B, C, HP, WP = 2, 96, 192, 192
H, W = 384, 384
NPLANE = B * C
NUP = HP * WP
MOUT = H * W
HALF = MOUT // 2
NTOT = NPLANE * NUP
WLEN = 19968
WOFF = NUP - WLEN
CH = 4992
NCHUNK = WLEN // CH
VPC = CH // 16

NC, NS = 2, 16
NW = NC * NS
PPW = NPLANE // NW
HPW = 2 * PPW

_mesh = plsc.VectorSubcoreMesh(core_axis_name="c", subcore_axis_name="s")


@functools.partial(
    pl.kernel,
    out_type=jax.ShapeDtypeStruct((NPLANE * MOUT,), jnp.float32),
    mesh=_mesh,
    scratch_types=[
        pltpu.VMEM((HALF,), jnp.float32),
        pltpu.VMEM((CH + 16,), jnp.int32),
        pltpu.VMEM((CH + 16,), jnp.int32),
        pltpu.VMEM((CH,), jnp.float32),
        pltpu.VMEM((CH,), jnp.float32),
        pltpu.SemaphoreType.DMA,
        pltpu.SemaphoreType.DMA,
        pltpu.SemaphoreType.DMA,
    ],
    compiler_params=pltpu.CompilerParams(needs_layout_passes=False),
)
def _scatter_sorted(sk_hbm, sv_hbm, out_hbm, buf, kv0, kv1, vv0, vv1, s0, s1, so):
  wid = lax.axis_index("s") * NC + lax.axis_index("c")
  zeros16 = jnp.zeros((16,), jnp.float32)
  lane = jax.lax.iota(jnp.int32, 16)
  sems = (s0, s1)
  kvs = (kv0, kv1)
  vvs = (vv0, vv1)

  def window_start(g):
    plane = wid * PPW + lax.shift_right_logical(g, 1)
    half = lax.bitwise_and(g, 1)
    return plane * NUP + half * WOFF

  def issue(g, c):
    b = c % 2
    start = window_start(g) + c * CH
    pltpu.make_async_copy(sk_hbm.at[pl.ds(start, CH)],
                          kvs[b].at[pl.ds(0, CH)], sems[b]).start()
    la = jnp.minimum(start + CH, NTOT - 16)
    pltpu.make_async_copy(sk_hbm.at[pl.ds(la, 16)],
                          kvs[b].at[pl.ds(CH, 16)], sems[b]).start()
    pltpu.make_async_copy(sv_hbm.at[pl.ds(start, CH)],
                          vvs[b], sems[b]).start()

  def wait_chunk(g, c):
    b = c % 2
    start = window_start(g) + c * CH
    pltpu.make_async_copy(sk_hbm.at[pl.ds(start, CH)],
                          kvs[b].at[pl.ds(0, CH)], sems[b]).wait()
    la = jnp.minimum(start + CH, NTOT - 16)
    pltpu.make_async_copy(sk_hbm.at[pl.ds(la, 16)],
                          kvs[b].at[pl.ds(CH, 16)], sems[b]).wait()
    pltpu.make_async_copy(sv_hbm.at[pl.ds(start, CH)],
                          vvs[b], sems[b]).wait()

  def out_copy(g):
    plane = wid * PPW + lax.shift_right_logical(g, 1)
    half = lax.bitwise_and(g, 1)
    kbase = plane * MOUT + half * HALF
    return pltpu.make_async_copy(buf, out_hbm.at[pl.ds(kbase, HALF)], so)

  issue(0, 0)
  issue(0, 1)

  @pl.loop(0, HPW)
  def _halfplane(g):
    plane = wid * PPW + lax.shift_right_logical(g, 1)
    half = lax.bitwise_and(g, 1)
    wstart = plane * NUP + half * WOFF
    kbase = plane * MOUT + half * HALF

    @pl.when(g > 0)
    def _():
      out_copy(g - 1).wait()

    @pl.loop(0, HALF // 16)
    def _zero(i):
      buf[pl.ds(i * 16, 16)] = zeros16

    for c in range(NCHUNK):
      b = c % 2
      wait_chunk(g, c)

      @pl.loop(0, VPC)
      def _vec(i):
        k0 = kvs[b][pl.ds(i * 16, 16)]
        k1 = kvs[b][pl.ds(i * 16 + 1, 16)]
        v = vvs[b][pl.ds(i * 16, 16)]
        local = k0 - kbase
        inr = plsc.bitcast(local, jnp.uint32) < jnp.uint32(HALF)
        pos = (wstart + c * CH + i * 16) + lane
        keep = (k0 != k1) | (pos == NTOT - 1)
        m = inr & keep
        plsc.store_scatter(buf, [local], v, mask=m)

      if c + 2 < NCHUNK:
        issue(g, c + 2)

    @pl.when(g < HPW - 1)
    def _():
      issue(g + 1, 0)
      issue(g + 1, 1)

    out_copy(g).start()

  out_copy(HPW - 1).wait()


def kernel(f, provenance):
  plane_off = (jnp.arange(NPLANE, dtype=jnp.int32) * MOUT).reshape(B, C, 1)
  keys = (provenance.reshape(B, C, NUP) + plane_off).reshape(-1)
  sk, sv = lax.sort((keys, f.reshape(-1)), dimension=0, is_stable=False,
                    num_keys=1)
  out = _scatter_sorted(sk, sv)
  return out.reshape(B, C, H, W)


# speedup vs baseline: 4.5197x; 1.0316x over previous
import functools
import jax
import jax.numpy as jnp
from jax import lax
from jax.experimental import pallas as pl
from jax.experimental.pallas import tpu as pltpu, tpu_sc as plsc

B, C, HP, WP = 2, 96, 192, 192
H, W = 384, 384
NPLANE = B * C
NUP = HP * WP
MOUT = H * W
HALF = MOUT // 2
NTOT = NPLANE * NUP
WLEN = 19968
WOFF = NUP - WLEN
CH = 4992
NCHUNK = WLEN // CH
VPC = CH // 16

NC, NS = 2, 16
NW = NC * NS
PPW = NPLANE // NW
HPW = 2 * PPW

_mesh = plsc.VectorSubcoreMesh(core_axis_name="c", subcore_axis_name="s")


@functools.partial(
    pl.kernel,
    out_type=jax.ShapeDtypeStruct((NPLANE * MOUT,), jnp.float32),
    mesh=_mesh,
    scratch_types=[
        pltpu.VMEM((HALF,), jnp.float32),
        pltpu.VMEM((CH + 16,), jnp.int32),
        pltpu.VMEM((CH + 16,), jnp.int32),
        pltpu.VMEM((CH,), jnp.float32),
        pltpu.VMEM((CH,), jnp.float32),
        pltpu.SemaphoreType.DMA,
        pltpu.SemaphoreType.DMA,
        pltpu.SemaphoreType.DMA,
    ],
    compiler_params=pltpu.CompilerParams(needs_layout_passes=False),
)
def _scatter_sorted(sk_hbm, sv_hbm, out_hbm, buf, kv0, kv1, vv0, vv1, s0, s1, so):
  wid = lax.axis_index("s") * NC + lax.axis_index("c")
  zeros16 = jnp.zeros((16,), jnp.float32)
  lane = jax.lax.iota(jnp.int32, 16)
  sems = (s0, s1)
  kvs = (kv0, kv1)
  vvs = (vv0, vv1)

  def window_start(g):
    plane = wid * PPW + lax.shift_right_logical(g, 1)
    half = lax.bitwise_and(g, 1)
    return plane * NUP + half * WOFF

  def issue(g, c):
    b = c % 2
    start = window_start(g) + c * CH
    pltpu.make_async_copy(sk_hbm.at[pl.ds(start, CH)],
                          kvs[b].at[pl.ds(0, CH)], sems[b]).start()
    la = jnp.minimum(start + CH, NTOT - 16)
    pltpu.make_async_copy(sk_hbm.at[pl.ds(la, 16)],
                          kvs[b].at[pl.ds(CH, 16)], sems[b]).start()
    pltpu.make_async_copy(sv_hbm.at[pl.ds(start, CH)],
                          vvs[b], sems[b]).start()

  def wait_chunk(g, c):
    b = c % 2
    start = window_start(g) + c * CH
    pltpu.make_async_copy(sk_hbm.at[pl.ds(start, CH)],
                          kvs[b].at[pl.ds(0, CH)], sems[b]).wait()
    la = jnp.minimum(start + CH, NTOT - 16)
    pltpu.make_async_copy(sk_hbm.at[pl.ds(la, 16)],
                          kvs[b].at[pl.ds(CH, 16)], sems[b]).wait()
    pltpu.make_async_copy(sv_hbm.at[pl.ds(start, CH)],
                          vvs[b], sems[b]).wait()

  def out_copy(g):
    plane = wid * PPW + lax.shift_right_logical(g, 1)
    half = lax.bitwise_and(g, 1)
    kbase = plane * MOUT + half * HALF
    return pltpu.make_async_copy(buf, out_hbm.at[pl.ds(kbase, HALF)], so)

  issue(0, 0)
  issue(0, 1)

  @pl.loop(0, HPW)
  def _halfplane(g):
    plane = wid * PPW + lax.shift_right_logical(g, 1)
    half = lax.bitwise_and(g, 1)
    wstart = plane * NUP + half * WOFF
    kbase = plane * MOUT + half * HALF

    @pl.when(g > 0)
    def _():
      out_copy(g - 1).wait()

    @pl.loop(0, HALF // 16, unroll=8)
    def _zero(i):
      buf[pl.ds(i * 16, 16)] = zeros16

    for c in range(NCHUNK):
      b = c % 2
      wait_chunk(g, c)

      @pl.loop(0, VPC, unroll=4)
      def _vec(i):
        k0 = kvs[b][pl.ds(i * 16, 16)]
        k1 = kvs[b][pl.ds(i * 16 + 1, 16)]
        v = vvs[b][pl.ds(i * 16, 16)]
        local = k0 - kbase
        inr = plsc.bitcast(local, jnp.uint32) < jnp.uint32(HALF)
        pos = (wstart + c * CH + i * 16) + lane
        keep = (k0 != k1) | (pos == NTOT - 1)
        m = inr & keep
        plsc.store_scatter(buf, [local], v, mask=m)

      if c + 2 < NCHUNK:
        issue(g, c + 2)

    @pl.when(g < HPW - 1)
    def _():
      issue(g + 1, 0)
      issue(g + 1, 1)

    out_copy(g).start()

  out_copy(HPW - 1).wait()


def kernel(f, provenance):
  plane_off = (jnp.arange(NPLANE, dtype=jnp.int32) * MOUT).reshape(B, C, 1)
  keys = (provenance.reshape(B, C, NUP) + plane_off).reshape(-1)
  sk, sv = lax.sort((keys, f.reshape(-1)), dimension=0, is_stable=False,
                    num_keys=1)
  out = _scatter_sorted(sk, sv)
  return out.reshape(B, C, H, W)
